# Initial kernel scaffold; baseline (speedup 1.0000x reference)
import jax
import jax.numpy as jnp
from jax.experimental import pallas as pl


def kernel(x_l, x_t, x_a, loc_table, tim_table, act_table, W1, b1, W2, b2, W3, b3, W4, b4, W5, b5):
    raise NotImplementedError("write your pallas kernel here")


# speedup vs baseline: 3.8849x; 3.8849x over previous
import functools

import jax
import jax.numpy as jnp
from jax import lax
from jax.experimental import pallas as pl
from jax.experimental.pallas import tpu as pltpu
from jax.experimental.pallas import tpu_sc as plsc

_B, _L = 4096, 50
_N = _B * _L
_LOC_DIM, _TIM_DIM, _ACT_DIM = 64, 32, 32
_LOC_W = _LOC_DIM // 2
_HID = 256
_TPAD = 304
_APAD = 16

_NC, _NS = 2, 16
_NW = _NC * _NS
_RPW = _N // _NW
_CH = 128
_NCH = _RPW // _CH
_GPB = 5
_RPB = _CH * _GPB
_NB = _RPW // _RPB

_BK = 8192


def _sc_body(xl_hbm, xtpk_hbm, loc_hbm, lout, xto, il, xv, r0, r1, sem0, sem1):
    wid = lax.axis_index("s") * _NC + lax.axis_index("c")
    pltpu.sync_copy(xl_hbm.at[wid], il)
    pltpu.sync_copy(xtpk_hbm.at[wid], xv)
    pltpu.sync_copy(xv, xto.at[pl.ds(wid * _RPW, _RPW)])

    bufs = ((r0, sem0), (r1, sem1))

    def issue(b, which):
        r, sem = bufs[which]
        for j in range(_GPB):
            ci = _GPB * b + j
            pltpu.async_copy(loc_hbm.at[il.at[ci]], r.at[pl.ds(j * _CH, _CH)], sem)

    def drain_and_writeback(b, which):
        r, sem = bufs[which]
        base = wid * _RPW + b * _RPB
        pltpu.make_async_copy(lout.at[pl.ds(base, _RPB)], r, sem).wait()
        pltpu.sync_copy(r, lout.at[pl.ds(base, _RPB)])

    issue(0, 0)
    for b in range(1, _NB):
        issue(b, b % 2)
        drain_and_writeback(b - 1, (b - 1) % 2)
    drain_and_writeback(_NB - 1, (_NB - 1) % 2)


def _sc_gather(xl, xtpk, loc_packed):
    mesh = plsc.VectorSubcoreMesh(core_axis_name="c", subcore_axis_name="s")
    kern = pl.kernel(
        _sc_body,
        out_type=(jax.ShapeDtypeStruct((_N, _LOC_W), jnp.int32),
                  jax.ShapeDtypeStruct((_N, 1), jnp.int32)),
        mesh=mesh,
        scratch_types=[
            pltpu.VMEM((_NCH, _CH), jnp.int32),
            pltpu.VMEM((_RPW, 1), jnp.int32),
            pltpu.VMEM((_RPB, _LOC_W), jnp.int32),
            pltpu.VMEM((_RPB, _LOC_W), jnp.int32),
            pltpu.SemaphoreType.DMA,
            pltpu.SemaphoreType.DMA,
        ],
        compiler_params=pltpu.CompilerParams(use_tc_tiling_on_sc=False),
    )
    return kern(xl, xtpk, loc_packed)


def _mlp_body(lp_ref, xtpk_ref, wt, wa, w1, b1, w2, b2, w3, b3,
              w4, b4, w5, b5, o_ref):
    f32 = jnp.float32
    bf = jnp.bfloat16
    lp = lp_ref[...]
    lo = lax.bitcast_convert_type(jnp.left_shift(lp, 16), f32).astype(bf)
    hi = lax.bitcast_convert_type(
        jnp.bitwise_and(lp, jnp.int32(-65536)), f32).astype(bf)
    xtpk = xtpk_ref[...]
    xt = jnp.right_shift(xtpk, 4)
    xa = jnp.bitwise_and(xtpk, 15)
    oht = (lax.broadcasted_iota(jnp.int32, (_BK, _TPAD), 1) == xt).astype(bf)
    oha = (lax.broadcasted_iota(jnp.int32, (_BK, _APAD), 1) == xa).astype(bf)
    temb = jnp.dot(oht, wt[...], preferred_element_type=f32).astype(bf)
    aemb = jnp.dot(oha, wa[...], preferred_element_type=f32).astype(bf)
    x = jnp.concatenate([lo, hi, temb, aemb], axis=1)
    h = jnp.dot(x, w1[...], preferred_element_type=f32)
    h = jnp.maximum(h.astype(bf) + b1[...], 0)
    h = jnp.maximum(jnp.dot(h, w2[...], preferred_element_type=f32).astype(bf) + b2[...], 0)
    h = jnp.maximum(jnp.dot(h, w3[...], preferred_element_type=f32).astype(bf) + b3[...], 0)
    h = jnp.maximum(jnp.dot(h, w4[...], preferred_element_type=f32) + b4[...], 0.0)
    z = jnp.sum(h.reshape(_BK // 128, 128, _HID) * w5[...], axis=2) + b5[...]
    o_ref[...] = 1.0 / (1.0 + jnp.exp(-z))


def _mlp(lemb, xtpk, wt, wa, W1, b1, W2, b2, W3, b3, W4, b4, w5t, b5):
    full = lambda shape: pl.BlockSpec(shape, lambda i: tuple(0 for _ in shape))
    return pl.pallas_call(
        _mlp_body,
        grid=(_N // _BK,),
        in_specs=[
            pl.BlockSpec((_BK, _LOC_W), lambda i: (i, 0)),
            pl.BlockSpec((_BK, 1), lambda i: (i, 0)),
            full((_TPAD, _TIM_DIM)),
            full((_APAD, _ACT_DIM)),
            full((_LOC_DIM + _TIM_DIM + _ACT_DIM, _HID)),
            full((1, _HID)),
            full((_HID, _HID)),
            full((1, _HID)),
            full((_HID, _HID)),
            full((1, _HID)),
            full((_HID, _HID)),
            full((1, _HID)),
            full((1, 1, _HID)),
            full((1, 1)),
        ],
        out_specs=pl.BlockSpec((_BK // 128, 128), lambda i: (i, 0)),
        out_shape=jax.ShapeDtypeStruct((_N // 128, 128), jnp.float32),
        compiler_params=pltpu.CompilerParams(
            dimension_semantics=("arbitrary",),
        ),
    )(lemb, xtpk, wt, wa, W1, b1, W2, b2, W3, b3, W4, b4, w5t, b5)


def kernel(x_l, x_t, x_a, loc_table, tim_table, act_table,
           W1, b1, W2, b2, W3, b3, W4, b4, W5, b5):
    bf = jnp.bfloat16
    loc_packed = lax.bitcast_convert_type(
        loc_table.astype(bf).reshape(-1, _LOC_W, 2), jnp.int32)
    xl = x_l.reshape(_NW, _NCH, _CH)
    xtpk = (x_t * 16 + x_a).reshape(_NW, _RPW, 1)
    lemb, xto = _sc_gather(xl, xtpk, loc_packed)

    w1perm = jnp.concatenate(
        [W1[0:_LOC_DIM:2], W1[1:_LOC_DIM:2], W1[_LOC_DIM:]], axis=0).astype(bf)
    wt = jnp.pad(tim_table, ((0, _TPAD - tim_table.shape[0]), (0, 0))).astype(bf)
    wa = jnp.pad(act_table, ((0, _APAD - act_table.shape[0]), (0, 0))).astype(bf)

    out = _mlp(
        lemb, xto, wt, wa, w1perm,
        b1.reshape(1, _HID).astype(bf), W2.astype(bf),
        b2.reshape(1, _HID).astype(bf), W3.astype(bf),
        b3.reshape(1, _HID).astype(bf), W4.astype(bf),
        b4.reshape(1, _HID), W5.reshape(1, 1, _HID), b5.reshape(1, 1),
    )
    return out.reshape(_B, _L, 1)
